# baseline (speedup 1.0000x reference)
import jax
import jax.numpy as jnp
from jax.experimental import pallas as pl


def _body(x_ref, pos_ref, o_ref):
    o_ref[...] = x_ref[...] + pos_ref[...][None]


def kernel(x, pos_table):
    batch, maxlen, embed = x.shape
    bb = 32
    return pl.pallas_call(
        _body,
        grid=(batch // bb,),
        in_specs=[
            pl.BlockSpec((bb, maxlen, embed), lambda i: (i, 0, 0)),
            pl.BlockSpec((maxlen, embed), lambda i: (0, 0)),
        ],
        out_specs=pl.BlockSpec((bb, maxlen, embed), lambda i: (i, 0, 0)),
        out_shape=jax.ShapeDtypeStruct(x.shape, x.dtype),
    )(x, pos_table)


# speedup vs baseline: 1.0328x; 1.0328x over previous
import jax
import jax.numpy as jnp
from jax.experimental import pallas as pl


def _body(x_ref, pos_ref, o_ref):
    o_ref[...] = x_ref[...] + pos_ref[...][None]


def kernel(x, pos_table):
    batch, maxlen, embed = x.shape
    bb = 64
    return pl.pallas_call(
        _body,
        grid=(batch // bb,),
        in_specs=[
            pl.BlockSpec((bb, maxlen, embed), lambda i: (i, 0, 0)),
            pl.BlockSpec((maxlen, embed), lambda i: (0, 0)),
        ],
        out_specs=pl.BlockSpec((bb, maxlen, embed), lambda i: (i, 0, 0)),
        out_shape=jax.ShapeDtypeStruct(x.shape, x.dtype),
    )(x, pos_table)
